# baseline (speedup 1.0000x reference)
import functools

import jax
import jax.numpy as jnp
from jax import lax
from jax.experimental import pallas as pl
from jax.experimental.pallas import tpu as pltpu
from jax.experimental.pallas import tpu_sc as plsc

N0 = 50000
NP = 50176
E0 = 1600000
ROWS = 12544
EP = ROWS * 128
F0 = 58
FP = 64
HALF = 32
NTILES = 16
NSL = NP // NTILES
RPT = ROWS // NTILES
RPW = ROWS // (2 * NTILES)
NSB = RPT // 8
CH = 112

_MESH = plsc.VectorSubcoreMesh(core_axis_name="c", subcore_axis_name="s")
_PARAMS = pltpu.CompilerParams(use_tc_tiling_on_sc=False,
                               needs_layout_passes=False)
_f32 = jnp.float32


def _zero_vmem_1d(ref, n):
    def zf(i, _):
        ref[pl.ds(i * 16, 16)] = jnp.zeros((16,), _f32)
        return 0
    lax.fori_loop(0, n // 16, zf, 0)


def _scale_chunk_rows(buf, dv_v, off):
    def rowloop16(g, _):
        d16 = dv_v[pl.ds(off + g * 16, 16)]
        for k in range(16):
            sc = d16[k]
            r = g * 16 + k
            for jj in range(HALF // 16):
                buf[r, pl.ds(jj * 16, 16)] = buf[r, pl.ds(jj * 16, 16)] * sc
        return 0
    lax.fori_loop(0, CH // 16, rowloop16, 0)


def _newton_rsqrt_chunk(dg_v, dv_v):
    def newton(i, _):
        sl = pl.ds(i * 16, 16)
        d = dg_v[sl]
        bits = lax.bitcast_convert_type(d, jnp.int32)
        bits = 0x5F3759DF - lax.shift_right_logical(bits, 1)
        y = lax.bitcast_convert_type(bits, _f32)
        for _it in range(3):
            y = y * (1.5 - 0.5 * d * y * y)
        dv_v[sl] = jnp.where(d > 0.5, y, 0.0)
        return 0
    lax.fori_loop(0, CH // 16, newton, 0)


@functools.partial(
    pl.kernel,
    mesh=_MESH,
    compiler_params=_PARAMS,
    out_type=(
        jax.ShapeDtypeStruct((2 * NP,), _f32),
        jax.ShapeDtypeStruct((2, NP, HALF), _f32),
        jax.ShapeDtypeStruct((2, NP, HALF), _f32),
        jax.ShapeDtypeStruct((2, NP, HALF), _f32),
        jax.ShapeDtypeStruct((2, NP, HALF), _f32),
        jax.ShapeDtypeStruct((2, NP, HALF), _f32),
        jax.ShapeDtypeStruct((2, NP, HALF), _f32),
    ),
    scratch_types=[
        pltpu.VMEM((3, 8, 128), jnp.int32),
        pltpu.VMEM((3, 8, 128), jnp.int32),
        pltpu.VMEM((4, 128, HALF), _f32),
        pltpu.VMEM((CH, HALF), _f32),
        pltpu.VMEM((CH,), _f32),
        pltpu.VMEM((CH,), _f32),
        pltpu.VMEM((128,), _f32),
        pltpu.VMEM_SHARED((NP, HALF), _f32),
        pltpu.VMEM_SHARED((NP,), _f32),
        pltpu.SemaphoreType.DMA,
        pltpu.SemaphoreType.DMA,
        pltpu.SemaphoreType.DMA,
        pltpu.SemaphoreType.DMA,
        pltpu.SemaphoreType.DMA,
        pltpu.SemaphoreType.DMA,
        pltpu.SemaphoreType.DMA,
        pltpu.SemaphoreType.DMA,
        pltpu.SemaphoreType.DMA,
    ],
)
def _layer1_kernel(src_hbm, dst_hbm, x2,
                   dinv2, t0, h1, t1, h2, t2, h3,
                   isrc, idst, rows, wb_v, dvc_v, dgc_v, ones_v,
                   acc_sp, deg_sp,
                   isem, gs0, gs1, gs2, gs3, ss0, ss1, ss2, ss3):
    c = lax.axis_index("c")
    s = lax.axis_index("s")
    gsem = (gs0, gs1, gs2, gs3)
    ssem = (ss0, ss1, ss2, ss3)
    base = s * RPT
    r0 = s * NSL

    _zero_vmem_1d(dgc_v, CH)
    for i in range(8):
        ones_v[pl.ds(i * 16, 16)] = jnp.ones((16,), _f32)

    def zdeg(i, _):
        pltpu.sync_copy(dgc_v, deg_sp.at[pl.ds(r0 + i * CH, CH)])
        return 0
    lax.fori_loop(0, NSL // CH, zdeg, 0)
    plsc.subcore_barrier()

    pltpu.sync_copy(dst_hbm.at[pl.ds(base, 8)], idst.at[0])

    def dgroup(gi, b):
        @pl.when(gi >= 1)
        def _():
            pltpu.make_async_copy(dst_hbm.at[pl.ds(0, 8)], idst.at[b],
                                  isem).wait()

        @pl.when(gi >= 2)
        def _():
            for j in range(8):
                pltpu.make_async_copy(
                    ones_v, deg_sp.at[pl.ds(0, 128)],
                    ssem[(b + 1) % 3]).wait()

        @pl.when(gi + 1 < NSB)
        def _():
            pltpu.async_copy(dst_hbm.at[pl.ds(base + (gi + 1) * 8, 8)],
                             idst.at[(b + 1) % 3], isem)

        for j in range(8):
            pltpu.async_copy(ones_v, deg_sp.at[idst.at[b, j]], ssem[b],
                             add=True)

    def douter(g3, _):
        for b in range(3):
            dgroup(g3 * 3 + b, b)
        return 0

    lax.fori_loop(0, NSB // 3, douter, 0)
    dgroup(NSB - 2, 0)
    dgroup(NSB - 1, 1)
    for b in (0, 1):
        for j in range(8):
            pltpu.make_async_copy(ones_v, deg_sp.at[pl.ds(0, 128)],
                                  ssem[b]).wait()
    plsc.subcore_barrier()

    def prep_chunk(i, _):
        rr = r0 + i * CH
        pltpu.sync_copy(deg_sp.at[pl.ds(rr, CH)], dgc_v)
        _newton_rsqrt_chunk(dgc_v, dvc_v)
        pltpu.sync_copy(dvc_v, dinv2.at[pl.ds(c * NP + rr, CH)])
        pltpu.sync_copy(x2.at[c].at[pl.ds(rr, CH)], wb_v)
        _scale_chunk_rows(wb_v, dvc_v, 0)
        pltpu.sync_copy(wb_v, t0.at[c].at[pl.ds(rr, CH)])
        return 0
    lax.fori_loop(0, NSL // CH, prep_chunk, 0)
    plsc.subcore_barrier()

    def edge_pipeline(tsrc):
        pltpu.sync_copy(src_hbm.at[pl.ds(base, 8)], isrc.at[0])
        pltpu.sync_copy(dst_hbm.at[pl.ds(base, 8)], idst.at[0])

        def super_block(g, b):
            pb = (b + 2) % 3
            nxt = (b + 1) % 3

            @pl.when(g >= 1)
            def _():
                pltpu.make_async_copy(src_hbm.at[pl.ds(0, 8)], isrc.at[b],
                                      isem).wait()
                pltpu.make_async_copy(src_hbm.at[pl.ds(0, 8)], idst.at[b],
                                      isem).wait()

            @pl.when(g + 1 < NSB)
            def _():
                rr = base + (g + 1) * 8
                pltpu.async_copy(src_hbm.at[pl.ds(rr, 8)], isrc.at[nxt],
                                 isem)
                pltpu.async_copy(dst_hbm.at[pl.ds(rr, 8)], idst.at[nxt],
                                 isem)

            for k in range(8):
                j = g * 8 + k
                s4 = k % 4

                @pl.when(j >= 4)
                def _():
                    pltpu.make_async_copy(
                        rows.at[s4], acc_sp.at[pl.ds(0, 128)],
                        ssem[s4]).wait()

                pltpu.async_copy(tsrc.at[isrc.at[b, k]], rows.at[s4],
                                 gsem[s4])

                s2 = (k - 2) % 4

                @pl.when(j >= 2)
                def _():
                    pltpu.make_async_copy(
                        tsrc.at[pl.ds(0, 128)], rows.at[s2],
                        gsem[s2]).wait()
                    ib = idst.at[b, k - 2] if k >= 2 else idst.at[pb, k + 6]
                    pltpu.async_copy(rows.at[s2], acc_sp.at[ib], ssem[s2],
                                     add=True)

        def outer(g3, _):
            for b in range(3):
                super_block(g3 * 3 + b, b)
            return 0

        lax.fori_loop(0, NSB // 3, outer, 0)
        super_block(NSB - 2, 0)
        super_block(NSB - 1, 1)
        for s2, kk in ((2, 6), (3, 7)):
            pltpu.make_async_copy(
                tsrc.at[pl.ds(0, 128)], rows.at[s2], gsem[s2]).wait()
            pltpu.async_copy(rows.at[s2], acc_sp.at[idst.at[1, kk]],
                             ssem[s2], add=True)
        for s4 in range(4):
            pltpu.make_async_copy(
                rows.at[s4], acc_sp.at[pl.ds(0, 128)], ssem[s4]).wait()

    def hop(tsrc_all, h_out, t_out):
        def zrow(r, _):
            for jj in range(HALF // 16):
                wb_v[r, pl.ds(jj * 16, 16)] = jnp.zeros((16,), _f32)
            return 0
        lax.fori_loop(0, CH, zrow, 0)

        def zc(i, _):
            pltpu.sync_copy(wb_v, acc_sp.at[pl.ds(r0 + i * CH, CH)])
            return 0
        lax.fori_loop(0, NSL // CH, zc, 0)
        plsc.subcore_barrier()

        edge_pipeline(tsrc_all.at[c])
        plsc.subcore_barrier()

        def wchunk(i, _):
            rr = r0 + i * CH
            pltpu.sync_copy(acc_sp.at[pl.ds(rr, CH)], wb_v)
            pltpu.sync_copy(dinv2.at[pl.ds(c * NP + rr, CH)], dvc_v)
            _scale_chunk_rows(wb_v, dvc_v, 0)
            pltpu.sync_copy(wb_v, h_out.at[c].at[pl.ds(rr, CH)])
            if t_out is not None:
                _scale_chunk_rows(wb_v, dvc_v, 0)
                pltpu.sync_copy(wb_v, t_out.at[c].at[pl.ds(rr, CH)])
            return 0
        lax.fori_loop(0, NSL // CH, wchunk, 0)
        plsc.subcore_barrier()

    hop(t0, h1, t1)
    hop(t1, h2, t2)
    hop(t2, h3, None)


@functools.partial(
    pl.kernel,
    mesh=_MESH,
    compiler_params=_PARAMS,
    out_type=jax.ShapeDtypeStruct((NP,), _f32),
    scratch_types=[
        pltpu.VMEM((NP,), _f32),
        pltpu.VMEM((NSL,), _f32),
        pltpu.VMEM((NSL,), _f32),
        pltpu.VMEM((NSL,), _f32),
        pltpu.VMEM((16,), _f32),
        pltpu.VMEM((3, 8, 128), jnp.int32),
        pltpu.VMEM((3, 8, 128), jnp.int32),
        pltpu.VMEM((3, 8, 128), _f32),
        pltpu.VMEM_SHARED((NP,), _f32),
        pltpu.VMEM_SHARED((NP,), _f32),
        pltpu.SemaphoreType.DMA,
        pltpu.SemaphoreType.DMA,
        pltpu.SemaphoreType.DMA,
        pltpu.SemaphoreType.DMA,
    ],
)
def _zloop_kernel(z3_hbm, z2_hbm, z1_hbm, z0_hbm, dinv_hbm,
                  src_hbm, dst_hbm, b2_hbm, out_hbm,
                  gt_v, nb_v, dv_v, p0_v, b2_v, isrc, idst, stage,
                  g_sp, acc_sp, isem, ss0, ss1, ss2):
    c = lax.axis_index("c")
    s = lax.axis_index("s")
    r0 = s * NSL
    base = s * RPT
    ssem = (ss0, ss1, ss2)
    pltpu.sync_copy(dinv_hbm.at[pl.ds(r0, NSL)], dv_v)
    pltpu.sync_copy(b2_hbm, b2_v)
    _zero_vmem_1d(p0_v, NSL)
    pltpu.sync_copy(p0_v, acc_sp.at[pl.ds(r0, NSL)])

    def edge_pipeline_z():
        pltpu.sync_copy(src_hbm.at[pl.ds(base, 8)], isrc.at[0])
        pltpu.sync_copy(dst_hbm.at[pl.ds(base, 8)], idst.at[0])

        def group(gi, b):
            @pl.when(gi >= 1)
            def _():
                pltpu.make_async_copy(src_hbm.at[pl.ds(0, 8)], isrc.at[b],
                                      isem).wait()
                pltpu.make_async_copy(src_hbm.at[pl.ds(0, 8)], idst.at[b],
                                      isem).wait()

            @pl.when(gi >= 2)
            def _():
                for j in range(8):
                    pltpu.make_async_copy(
                        stage.at[(b + 1) % 3, j], acc_sp.at[pl.ds(0, 128)],
                        ssem[(b + 1) % 3]).wait()

            @pl.when(gi + 1 < NSB)
            def _():
                rr = base + (gi + 1) * 8
                pltpu.async_copy(src_hbm.at[pl.ds(rr, 8)],
                                 isrc.at[(b + 1) % 3], isem)
                pltpu.async_copy(dst_hbm.at[pl.ds(rr, 8)],
                                 idst.at[(b + 1) % 3], isem)

            for j in range(8):
                for jj in range(8):
                    iv = isrc[b, j, pl.ds(jj * 16, 16)]
                    stage[b, j, pl.ds(jj * 16, 16)] = \
                        plsc.load_gather(gt_v, [iv])
            for j in range(8):
                pltpu.async_copy(stage.at[b, j], acc_sp.at[idst.at[b, j]],
                                 ssem[b], add=True)

        def outer(g3, _):
            for b in range(3):
                group(g3 * 3 + b, b)
            return 0

        lax.fori_loop(0, NSB // 3, outer, 0)
        group(NSB - 2, 0)
        group(NSB - 1, 1)
        for b in (0, 1):
            for j in range(8):
                pltpu.make_async_copy(
                    stage.at[b, j], acc_sp.at[pl.ds(0, 128)],
                    ssem[b]).wait()

    for z_hbm in (z3_hbm, z2_hbm, z1_hbm):
        pltpu.sync_copy(z_hbm.at[pl.ds(r0, NSL)], nb_v)
        pltpu.sync_copy(acc_sp.at[pl.ds(r0, NSL)], p0_v)

        def gcalc(i, _):
            sl = pl.ds(i * 16, 16)
            d = dv_v[sl]
            nb_v[sl] = d * (nb_v[sl] + d * p0_v[sl])
            return 0
        lax.fori_loop(0, NSL // 16, gcalc, 0)
        pltpu.sync_copy(nb_v, g_sp.at[pl.ds(r0, NSL)])
        _zero_vmem_1d(nb_v, NSL)
        pltpu.sync_copy(nb_v, acc_sp.at[pl.ds(r0, NSL)])
        plsc.subcore_barrier()
        pltpu.sync_copy(g_sp, gt_v)
        edge_pipeline_z()
        plsc.subcore_barrier()

    pltpu.sync_copy(z0_hbm.at[pl.ds(r0, NSL)], nb_v)
    pltpu.sync_copy(acc_sp.at[pl.ds(r0, NSL)], p0_v)
    b2vec = b2_v[pl.ds(0, 16)]
    b2s = b2vec[0]

    def fin(i, _):
        sl = pl.ds(i * 16, 16)
        nb_v[sl] = nb_v[sl] + dv_v[sl] * p0_v[sl] + b2s
        return 0
    lax.fori_loop(0, NSL // 16, fin, 0)

    @pl.when(c == 0)
    def _():
        pltpu.sync_copy(nb_v, out_hbm.at[pl.ds(r0, NSL)])


_RB = NP // 8


def _combine_body(x_ref, h1_ref, h2_ref, h3_ref,
                  w0_ref, wlo_ref, whi_ref, b1_ref, w2_ref, z_ref):
    acc = jnp.dot(h1_ref[0], wlo_ref[0], preferred_element_type=_f32)
    acc += jnp.dot(h1_ref[1], whi_ref[0], preferred_element_type=_f32)
    acc += jnp.dot(h2_ref[0], wlo_ref[1], preferred_element_type=_f32)
    acc += jnp.dot(h2_ref[1], whi_ref[1], preferred_element_type=_f32)
    acc += jnp.dot(h3_ref[0], wlo_ref[2], preferred_element_type=_f32)
    acc += jnp.dot(h3_ref[1], whi_ref[2], preferred_element_type=_f32)
    h = jnp.dot(x_ref[...], w0_ref[...], preferred_element_type=_f32)
    h = h + acc + b1_ref[...]
    h = jnp.maximum(h, 0.0)
    z_ref[...] = jnp.dot(h, w2_ref[...], preferred_element_type=_f32)


_combine_call = pl.pallas_call(
    _combine_body,
    grid=(8,),
    in_specs=[
        pl.BlockSpec((_RB, FP), lambda i: (i, 0)),
        pl.BlockSpec((2, _RB, HALF), lambda i: (0, i, 0)),
        pl.BlockSpec((2, _RB, HALF), lambda i: (0, i, 0)),
        pl.BlockSpec((2, _RB, HALF), lambda i: (0, i, 0)),
        pl.BlockSpec((FP, 128), lambda i: (0, 0)),
        pl.BlockSpec((3, HALF, 128), lambda i: (0, 0, 0)),
        pl.BlockSpec((3, HALF, 128), lambda i: (0, 0, 0)),
        pl.BlockSpec((1, 128), lambda i: (0, 0)),
        pl.BlockSpec((128, 4), lambda i: (0, 0)),
    ],
    out_specs=pl.BlockSpec((_RB, 4), lambda i: (i, 0)),
    out_shape=jax.ShapeDtypeStruct((NP, 4), _f32),
)


def _impl(x, edge_index, W1, b1, W2, b2):
    src = edge_index[0]
    dst = edge_index[1]
    padi = N0 + jnp.arange(EP - E0, dtype=jnp.int32) % (NP - N0)
    srcp = jnp.concatenate([src, padi]).reshape(ROWS, 128)
    dstp = jnp.concatenate([dst, padi]).reshape(ROWS, 128)
    xp = jnp.pad(x, ((0, NP - N0), (0, FP - F0)))
    x2 = jnp.stack([xp[:, :HALF], xp[:, HALF:]])

    W1p = jnp.pad(W1, ((0, 0), (0, FP - F0), (0, 0)))
    w0 = W1p[0]
    wlo = W1p[1:, :HALF, :]
    whi = W1p[1:, HALF:, :]
    b1r = b1.reshape(1, 128)
    w2c = jnp.transpose(W2[:, :, 0])
    b2p = jnp.pad(b2, (0, 15))

    dinv2, _t0, h1, _t1, h2, _t2, h3 = _layer1_kernel(srcp, dstp, x2)
    dinv_f = dinv2[:NP]

    z = _combine_call(xp, h1, h2, h3, w0, wlo, whi, b1r, w2c)

    out_full = _zloop_kernel(z[:, 3], z[:, 2], z[:, 1], z[:, 0],
                             dinv_f, srcp, dstp, b2p)
    return out_full[:N0][:, None]


kernel = jax.jit(_impl)


# speedup vs baseline: 1.0573x; 1.0573x over previous
import functools

import jax
import jax.numpy as jnp
from jax import lax
from jax.experimental import pallas as pl
from jax.experimental.pallas import tpu as pltpu
from jax.experimental.pallas import tpu_sc as plsc

N0 = 50000
NP = 50176
E0 = 1600000
ROWS = 12544
EP = ROWS * 128
F0 = 58
FP = 64
HALF = 32
NTILES = 16
NSL = NP // NTILES
RPT = ROWS // NTILES
RPW = ROWS // (2 * NTILES)
NSB = RPT // 8
CH = 112

_MESH = plsc.VectorSubcoreMesh(core_axis_name="c", subcore_axis_name="s")
_PARAMS = pltpu.CompilerParams(use_tc_tiling_on_sc=False,
                               needs_layout_passes=False)
_f32 = jnp.float32


def _zero_vmem_1d(ref, n):
    def zf(i, _):
        ref[pl.ds(i * 16, 16)] = jnp.zeros((16,), _f32)
        return 0
    lax.fori_loop(0, n // 16, zf, 0)


def _scale_chunk_rows(buf, dv_v, off):
    def rowloop16(g, _):
        d16 = dv_v[pl.ds(off + g * 16, 16)]
        for k in range(16):
            sc = d16[k]
            r = g * 16 + k
            for jj in range(HALF // 16):
                buf[r, pl.ds(jj * 16, 16)] = buf[r, pl.ds(jj * 16, 16)] * sc
        return 0
    lax.fori_loop(0, CH // 16, rowloop16, 0)


def _newton_rsqrt_chunk(dg_v, dv_v):
    def newton(i, _):
        sl = pl.ds(i * 16, 16)
        d = dg_v[sl]
        bits = lax.bitcast_convert_type(d, jnp.int32)
        bits = 0x5F3759DF - lax.shift_right_logical(bits, 1)
        y = lax.bitcast_convert_type(bits, _f32)
        for _it in range(3):
            y = y * (1.5 - 0.5 * d * y * y)
        dv_v[sl] = jnp.where(d > 0.5, y, 0.0)
        return 0
    lax.fori_loop(0, CH // 16, newton, 0)


@functools.partial(
    pl.kernel,
    mesh=_MESH,
    compiler_params=_PARAMS,
    out_type=(
        jax.ShapeDtypeStruct((2 * NP,), _f32),
        jax.ShapeDtypeStruct((2, NP, HALF), _f32),
        jax.ShapeDtypeStruct((2, NP, HALF), _f32),
        jax.ShapeDtypeStruct((2, NP, HALF), _f32),
        jax.ShapeDtypeStruct((2, NP, HALF), _f32),
        jax.ShapeDtypeStruct((2, NP, HALF), _f32),
        jax.ShapeDtypeStruct((2, NP, HALF), _f32),
    ),
    scratch_types=[
        pltpu.VMEM((3, 8, 128), jnp.int32),
        pltpu.VMEM((3, 8, 128), jnp.int32),
        pltpu.VMEM((4, 128, HALF), _f32),
        pltpu.VMEM((CH, HALF), _f32),
        pltpu.VMEM((CH,), _f32),
        pltpu.VMEM((CH,), _f32),
        pltpu.VMEM((128,), _f32),
        pltpu.VMEM_SHARED((NP, HALF), _f32),
        pltpu.VMEM_SHARED((NP,), _f32),
        pltpu.SemaphoreType.DMA,
        pltpu.SemaphoreType.DMA,
        pltpu.SemaphoreType.DMA,
        pltpu.SemaphoreType.DMA,
        pltpu.SemaphoreType.DMA,
        pltpu.SemaphoreType.DMA,
        pltpu.SemaphoreType.DMA,
        pltpu.SemaphoreType.DMA,
        pltpu.SemaphoreType.DMA,
    ],
)
def _layer1_kernel(src_hbm, dst_hbm, x2,
                   dinv2, t0, h1, t1, h2, t2, h3,
                   isrc, idst, rows, wb_v, dvc_v, dgc_v, ones_v,
                   acc_sp, deg_sp,
                   isem, gs0, gs1, gs2, gs3, ss0, ss1, ss2, ss3):
    c = lax.axis_index("c")
    s = lax.axis_index("s")
    gsem = (gs0, gs1, gs2, gs3)
    ssem = (ss0, ss1, ss2, ss3)
    base = s * RPT
    r0 = s * NSL

    _zero_vmem_1d(dgc_v, CH)
    for i in range(8):
        ones_v[pl.ds(i * 16, 16)] = jnp.ones((16,), _f32)

    def zdeg(i, _):
        pltpu.sync_copy(dgc_v, deg_sp.at[pl.ds(r0 + i * CH, CH)])
        return 0
    lax.fori_loop(0, NSL // CH, zdeg, 0)
    plsc.subcore_barrier()

    pltpu.sync_copy(dst_hbm.at[pl.ds(base, 8)], idst.at[0])

    def dgroup(gi, b):
        @pl.when(gi >= 1)
        def _():
            pltpu.make_async_copy(dst_hbm.at[pl.ds(0, 8)], idst.at[b],
                                  isem).wait()

        @pl.when(gi >= 2)
        def _():
            for j in range(8):
                pltpu.make_async_copy(
                    ones_v, deg_sp.at[pl.ds(0, 128)],
                    ssem[(b + 1) % 3]).wait()

        @pl.when(gi + 1 < NSB)
        def _():
            pltpu.async_copy(dst_hbm.at[pl.ds(base + (gi + 1) * 8, 8)],
                             idst.at[(b + 1) % 3], isem)

        for j in range(8):
            pltpu.async_copy(ones_v, deg_sp.at[idst.at[b, j]], ssem[b],
                             add=True)

    def douter(g3, _):
        for b in range(3):
            dgroup(g3 * 3 + b, b)
        return 0

    lax.fori_loop(0, NSB // 3, douter, 0)
    dgroup(NSB - 2, 0)
    dgroup(NSB - 1, 1)
    for b in (0, 1):
        for j in range(8):
            pltpu.make_async_copy(ones_v, deg_sp.at[pl.ds(0, 128)],
                                  ssem[b]).wait()
    plsc.subcore_barrier()

    def prep_chunk(i, _):
        rr = r0 + i * CH
        pltpu.sync_copy(deg_sp.at[pl.ds(rr, CH)], dgc_v)
        _newton_rsqrt_chunk(dgc_v, dvc_v)
        pltpu.sync_copy(dvc_v, dinv2.at[pl.ds(c * NP + rr, CH)])
        pltpu.sync_copy(x2.at[c].at[pl.ds(rr, CH)], wb_v)
        _scale_chunk_rows(wb_v, dvc_v, 0)
        pltpu.sync_copy(wb_v, t0.at[c].at[pl.ds(rr, CH)])
        return 0
    lax.fori_loop(0, NSL // CH, prep_chunk, 0)
    plsc.subcore_barrier()

    def edge_pipeline(tsrc):
        pltpu.sync_copy(src_hbm.at[pl.ds(base, 8)], isrc.at[0])
        pltpu.sync_copy(dst_hbm.at[pl.ds(base, 8)], idst.at[0])

        def super_block(g, b):
            pb = (b + 2) % 3
            nxt = (b + 1) % 3

            @pl.when(g >= 1)
            def _():
                pltpu.make_async_copy(src_hbm.at[pl.ds(0, 8)], isrc.at[b],
                                      isem).wait()
                pltpu.make_async_copy(src_hbm.at[pl.ds(0, 8)], idst.at[b],
                                      isem).wait()

            @pl.when(g + 1 < NSB)
            def _():
                rr = base + (g + 1) * 8
                pltpu.async_copy(src_hbm.at[pl.ds(rr, 8)], isrc.at[nxt],
                                 isem)
                pltpu.async_copy(dst_hbm.at[pl.ds(rr, 8)], idst.at[nxt],
                                 isem)

            for k in range(8):
                j = g * 8 + k
                s4 = k % 4

                @pl.when(j >= 4)
                def _():
                    pltpu.make_async_copy(
                        rows.at[s4], acc_sp.at[pl.ds(0, 128)],
                        ssem[s4]).wait()

                pltpu.async_copy(tsrc.at[isrc.at[b, k]], rows.at[s4],
                                 gsem[s4])

                s2 = (k - 2) % 4

                @pl.when(j >= 2)
                def _():
                    pltpu.make_async_copy(
                        tsrc.at[pl.ds(0, 128)], rows.at[s2],
                        gsem[s2]).wait()
                    ib = idst.at[b, k - 2] if k >= 2 else idst.at[pb, k + 6]
                    pltpu.async_copy(rows.at[s2], acc_sp.at[ib], ssem[s2],
                                     add=True)

        def outer(g3, _):
            for b in range(3):
                super_block(g3 * 3 + b, b)
            return 0

        lax.fori_loop(0, NSB // 3, outer, 0)
        super_block(NSB - 2, 0)
        super_block(NSB - 1, 1)
        for s2, kk in ((2, 6), (3, 7)):
            pltpu.make_async_copy(
                tsrc.at[pl.ds(0, 128)], rows.at[s2], gsem[s2]).wait()
            pltpu.async_copy(rows.at[s2], acc_sp.at[idst.at[1, kk]],
                             ssem[s2], add=True)
        for s4 in range(4):
            pltpu.make_async_copy(
                rows.at[s4], acc_sp.at[pl.ds(0, 128)], ssem[s4]).wait()

    def hop(tsrc_all, h_out, t_out):
        def zrow(r, _):
            for jj in range(HALF // 16):
                wb_v[r, pl.ds(jj * 16, 16)] = jnp.zeros((16,), _f32)
            return 0
        lax.fori_loop(0, CH, zrow, 0)

        def zc(i, _):
            pltpu.sync_copy(wb_v, acc_sp.at[pl.ds(r0 + i * CH, CH)])
            return 0
        lax.fori_loop(0, NSL // CH, zc, 0)
        plsc.subcore_barrier()

        edge_pipeline(tsrc_all.at[c])
        plsc.subcore_barrier()

        def wchunk(i, _):
            rr = r0 + i * CH
            pltpu.sync_copy(acc_sp.at[pl.ds(rr, CH)], wb_v)
            pltpu.sync_copy(dinv2.at[pl.ds(c * NP + rr, CH)], dvc_v)
            _scale_chunk_rows(wb_v, dvc_v, 0)
            pltpu.sync_copy(wb_v, h_out.at[c].at[pl.ds(rr, CH)])
            if t_out is not None:
                _scale_chunk_rows(wb_v, dvc_v, 0)
                pltpu.sync_copy(wb_v, t_out.at[c].at[pl.ds(rr, CH)])
            return 0
        lax.fori_loop(0, NSL // CH, wchunk, 0)
        plsc.subcore_barrier()

    hop(t0, h1, t1)
    hop(t1, h2, t2)
    hop(t2, h3, None)


@functools.partial(
    pl.kernel,
    mesh=_MESH,
    compiler_params=_PARAMS,
    out_type=jax.ShapeDtypeStruct((2 * NP,), _f32),
    scratch_types=[
        pltpu.VMEM((NP,), _f32),
        pltpu.VMEM((NSL,), _f32),
        pltpu.VMEM((NSL,), _f32),
        pltpu.VMEM((NSL,), _f32),
        pltpu.VMEM((NSL,), _f32),
        pltpu.VMEM((3, 8, 128), jnp.int32),
        pltpu.VMEM((3, 8, 128), jnp.int32),
        pltpu.VMEM((3, 8, 128), _f32),
        pltpu.VMEM_SHARED((NP,), _f32),
        pltpu.VMEM_SHARED((NP,), _f32),
        pltpu.SemaphoreType.DMA,
        pltpu.SemaphoreType.DMA,
        pltpu.SemaphoreType.DMA,
        pltpu.SemaphoreType.DMA,
    ],
)
def _zhop_kernel(z_hbm, pin_hbm, dinv_hbm, src_hbm, dst_hbm, pout,
                 gt_v, nb_v, dv_v, p0_v, p1_v, isrc, idst, stage,
                 g_sp, acc_sp, isem, ss0, ss1, ss2):
    c = lax.axis_index("c")
    s = lax.axis_index("s")
    wid = c * NTILES + s
    r0 = s * NSL
    pltpu.sync_copy(z_hbm.at[pl.ds(r0, NSL)], nb_v)
    pltpu.sync_copy(dinv_hbm.at[pl.ds(r0, NSL)], dv_v)
    pltpu.sync_copy(pin_hbm.at[pl.ds(r0, NSL)], p0_v)
    pltpu.sync_copy(pin_hbm.at[pl.ds(NP + r0, NSL)], p1_v)

    def gcalc(i, _):
        sl = pl.ds(i * 16, 16)
        d = dv_v[sl]
        nb_v[sl] = d * (nb_v[sl] + d * (p0_v[sl] + p1_v[sl]))
        return 0
    lax.fori_loop(0, NSL // 16, gcalc, 0)
    pltpu.sync_copy(nb_v, g_sp.at[pl.ds(r0, NSL)])
    _zero_vmem_1d(nb_v, NSL)
    pltpu.sync_copy(nb_v, acc_sp.at[pl.ds(r0, NSL)])
    plsc.subcore_barrier()

    pltpu.sync_copy(g_sp, gt_v)

    ssem = (ss0, ss1, ss2)
    NG = RPW // 8
    base = wid * RPW
    pltpu.sync_copy(src_hbm.at[pl.ds(base, 8)], isrc.at[0])
    pltpu.sync_copy(dst_hbm.at[pl.ds(base, 8)], idst.at[0])

    def group(gi, b):
        @pl.when(gi >= 1)
        def _():
            pltpu.make_async_copy(src_hbm.at[pl.ds(0, 8)], isrc.at[b],
                                  isem).wait()
            pltpu.make_async_copy(src_hbm.at[pl.ds(0, 8)], idst.at[b],
                                  isem).wait()

        @pl.when(gi >= 2)
        def _():
            for j in range(8):
                pltpu.make_async_copy(
                    stage.at[(b + 1) % 3, j], acc_sp.at[pl.ds(0, 128)],
                    ssem[(b + 1) % 3]).wait()

        @pl.when(gi + 1 < NG)
        def _():
            rr = base + (gi + 1) * 8
            pltpu.async_copy(src_hbm.at[pl.ds(rr, 8)],
                             isrc.at[(b + 1) % 3], isem)
            pltpu.async_copy(dst_hbm.at[pl.ds(rr, 8)],
                             idst.at[(b + 1) % 3], isem)

        for j in range(8):
            for jj in range(8):
                iv = isrc[b, j, pl.ds(jj * 16, 16)]
                stage[b, j, pl.ds(jj * 16, 16)] = \
                    plsc.load_gather(gt_v, [iv])
        for j in range(8):
            pltpu.async_copy(stage.at[b, j], acc_sp.at[idst.at[b, j]],
                             ssem[b], add=True)

    def outer(g3, _):
        for b in range(3):
            group(g3 * 3 + b, b)
        return 0

    lax.fori_loop(0, NG // 3, outer, 0)
    group(NG - 1, 0)
    for b in (0, 2):
        for j in range(8):
            pltpu.make_async_copy(stage.at[b, j], acc_sp.at[pl.ds(0, 128)],
                                  ssem[b]).wait()
    plsc.subcore_barrier()
    pltpu.sync_copy(acc_sp.at[pl.ds(r0, NSL)], nb_v)
    pltpu.sync_copy(nb_v, pout.at[pl.ds(c * NP + r0, NSL)])


_RB = NP // 8


def _combine_body(x_ref, h1_ref, h2_ref, h3_ref,
                  w0_ref, wlo_ref, whi_ref, b1_ref, w2_ref, z_ref):
    acc = jnp.dot(h1_ref[0], wlo_ref[0], preferred_element_type=_f32)
    acc += jnp.dot(h1_ref[1], whi_ref[0], preferred_element_type=_f32)
    acc += jnp.dot(h2_ref[0], wlo_ref[1], preferred_element_type=_f32)
    acc += jnp.dot(h2_ref[1], whi_ref[1], preferred_element_type=_f32)
    acc += jnp.dot(h3_ref[0], wlo_ref[2], preferred_element_type=_f32)
    acc += jnp.dot(h3_ref[1], whi_ref[2], preferred_element_type=_f32)
    h = jnp.dot(x_ref[...], w0_ref[...], preferred_element_type=_f32)
    h = h + acc + b1_ref[...]
    h = jnp.maximum(h, 0.0)
    z_ref[...] = jnp.dot(h, w2_ref[...], preferred_element_type=_f32)


_combine_call = pl.pallas_call(
    _combine_body,
    grid=(8,),
    in_specs=[
        pl.BlockSpec((_RB, FP), lambda i: (i, 0)),
        pl.BlockSpec((2, _RB, HALF), lambda i: (0, i, 0)),
        pl.BlockSpec((2, _RB, HALF), lambda i: (0, i, 0)),
        pl.BlockSpec((2, _RB, HALF), lambda i: (0, i, 0)),
        pl.BlockSpec((FP, 128), lambda i: (0, 0)),
        pl.BlockSpec((3, HALF, 128), lambda i: (0, 0, 0)),
        pl.BlockSpec((3, HALF, 128), lambda i: (0, 0, 0)),
        pl.BlockSpec((1, 128), lambda i: (0, 0)),
        pl.BlockSpec((128, 4), lambda i: (0, 0)),
    ],
    out_specs=pl.BlockSpec((_RB, 4), lambda i: (i, 0)),
    out_shape=jax.ShapeDtypeStruct((NP, 4), _f32),
)


def _final_body(z0_ref, p_ref, dinv_ref, b2_ref, out_ref):
    out_ref[...] = (z0_ref[...] + dinv_ref[...] * (p_ref[0] + p_ref[1])
                    + b2_ref[...])


_final_call = pl.pallas_call(
    _final_body,
    out_shape=jax.ShapeDtypeStruct((392, 128), _f32),
)


def _impl(x, edge_index, W1, b1, W2, b2):
    src = edge_index[0]
    dst = edge_index[1]
    padi = N0 + jnp.arange(EP - E0, dtype=jnp.int32) % (NP - N0)
    srcp = jnp.concatenate([src, padi]).reshape(ROWS, 128)
    dstp = jnp.concatenate([dst, padi]).reshape(ROWS, 128)
    xp = jnp.pad(x, ((0, NP - N0), (0, FP - F0)))
    x2 = jnp.stack([xp[:, :HALF], xp[:, HALF:]])

    W1p = jnp.pad(W1, ((0, 0), (0, FP - F0), (0, 0)))
    w0 = W1p[0]
    wlo = W1p[1:, :HALF, :]
    whi = W1p[1:, HALF:, :]
    b1r = b1.reshape(1, 128)
    w2c = jnp.transpose(W2[:, :, 0])
    b2r = b2.reshape(1, 1)

    dinv2, _t0, h1, _t1, h2, _t2, h3 = _layer1_kernel(srcp, dstp, x2)
    dinv_f = dinv2[:NP]

    z = _combine_call(xp, h1, h2, h3, w0, wlo, whi, b1r, w2c)

    zeros = jnp.zeros((2 * NP,), _f32)
    P3 = _zhop_kernel(z[:, 3], zeros, dinv_f, srcp, dstp)
    P2 = _zhop_kernel(z[:, 2], P3, dinv_f, srcp, dstp)
    P1 = _zhop_kernel(z[:, 1], P2, dinv_f, srcp, dstp)

    res = _final_call(z[:, 0].reshape(392, 128),
                      P1.reshape(2, 392, 128),
                      dinv_f.reshape(392, 128), b2r)
    return res.reshape(NP)[:N0][:, None]


kernel = jax.jit(_impl)
